# Initial kernel scaffold; baseline (speedup 1.0000x reference)
import functools

import jax
import jax.numpy as jnp
from jax import lax
from jax.experimental import pallas as pl
from jax.experimental.pallas import tpu as pltpu
from jax.experimental.pallas import tpu_sc as plsc

NB = 10000
EB = 320000
NH1 = 8
HID = 8
D1 = NH1 * HID
NC2 = 40
D2 = 48

CHUNK = 512
SUB = 128
NSUB = CHUNK // SUB
NCHUNKS = EB // CHUNK
NWORK = 32
ROWS_PT = NB // 16


def _leaky(x):
    return jnp.maximum(x, 0.2 * x)


def _tc1_body(x_ref, w1_ref, ams_ref, amd_ref, h1_ref, as_ref, ad_ref, g_ref):
    h = jnp.dot(x_ref[...], w1_ref[...], preferred_element_type=jnp.float32)
    h1_ref[...] = h
    a_s = jnp.dot(h, ams_ref[...], preferred_element_type=jnp.float32)
    a_d = jnp.dot(h, amd_ref[...], preferred_element_type=jnp.float32)
    as_ref[...] = a_s
    ad_ref[...] = a_d
    g_ref[...] = jnp.max(a_s, axis=0, keepdims=True)


def _tc2_body(dp_ref, dinv_ref):
    dinv_ref[...] = 1.0 / (dp_ref[0] + dp_ref[1] + 1e-16)


def _tc3_body(op_ref, w2_ref, as2_ref, ad2_ref, h2_ref, s2_ref, d2_ref, g2_ref):
    o = op_ref[0] + op_ref[1]
    hact = jnp.where(o > 0, o, jnp.expm1(o))
    h2 = jnp.dot(hact, w2_ref[...], preferred_element_type=jnp.float32)
    h2_ref[...] = h2
    s2 = jnp.dot(h2, as2_ref[...], preferred_element_type=jnp.float32)
    d2 = jnp.dot(h2, ad2_ref[...], preferred_element_type=jnp.float32)
    s2_ref[...] = s2
    d2_ref[...] = d2
    g2_ref[...] = jnp.max(s2, axis=0, keepdims=True)


def _tc5_body(op_ref, out_ref):
    o = op_ref[0] + op_ref[1]
    t = o[:, :NC2]
    m = jnp.max(t, axis=1, keepdims=True)
    lse = jnp.log(jnp.sum(jnp.exp(t - m), axis=1, keepdims=True))
    out_ref[...] = t - m - lse


def _wid_and_niter(nch):
    cid = lax.axis_index("c")
    sid = lax.axis_index("s")
    wid = sid * 2 + cid
    n_i = (nch - wid + NWORK - 1) // NWORK
    return cid, sid, wid, n_i


def _sc_a1_body(src_hbm, dst_hbm, as_hbm, ad_hbm, g_hbm,
                eexp_hbm, dpart_hbm,
                srcv, dstv, asv, adv, eev, gv, dsh):
    cid, sid, wid, n_i = _wid_and_niter(NCHUNKS)
    eevf = eev.reshape(CHUNK * NH1)
    asvf = asv.reshape(CHUNK * NH1)
    advf = adv.reshape(CHUNK * NH1)
    zero = jnp.zeros((16,), jnp.float32)

    def zb(j, _):
        eevf[pl.ds(16 * j, 16)] = zero
        return 0
    lax.fori_loop(0, CHUNK * NH1 // 16, zb, 0)
    pltpu.sync_copy(eev.at[pl.ds(0, CHUNK)], dsh.at[pl.ds(sid * ROWS_PT, CHUNK)])
    pltpu.sync_copy(eev.at[pl.ds(0, ROWS_PT - CHUNK)],
                    dsh.at[pl.ds(sid * ROWS_PT + CHUNK, ROWS_PT - CHUNK)])
    pltpu.sync_copy(g_hbm, gv)
    plsc.subcore_barrier()

    g = gv[...]

    def chunk_body(i, _):
        c = wid + NWORK * i
        pltpu.sync_copy(src_hbm.at[pl.ds(NSUB * c, NSUB)], srcv)
        pltpu.sync_copy(dst_hbm.at[pl.ds(NSUB * c, NSUB)], dstv)
        for k in range(NSUB):
            pltpu.sync_copy(as_hbm.at[srcv.at[k]], asv.at[pl.ds(SUB * k, SUB)])
            pltpu.sync_copy(ad_hbm.at[dstv.at[k]], adv.at[pl.ds(SUB * k, SUB)])

        def inner(j, _):
            s = asvf[pl.ds(16 * j, 16)]
            a = advf[pl.ds(16 * j, 16)]
            e = _leaky(s + a)
            cb = _leaky(g + a)
            eevf[pl.ds(16 * j, 16)] = jnp.exp(e - cb)
            return 0
        lax.fori_loop(0, CHUNK * NH1 // 16, inner, 0)

        for k in range(NSUB):
            pltpu.sync_copy(eev.at[pl.ds(SUB * k, SUB)], dsh.at[dstv.at[k]],
                            add=True)
        pltpu.sync_copy(eev, eexp_hbm.at[pl.ds(CHUNK * c, CHUNK)])
        return 0
    lax.fori_loop(0, n_i, chunk_body, 0)

    plsc.subcore_barrier()
    pltpu.sync_copy(dsh.at[pl.ds(sid * ROWS_PT, ROWS_PT)],
                    dpart_hbm.at[cid, pl.ds(sid * ROWS_PT, ROWS_PT)])


def _sc_b1_body(src_hbm, dst_hbm, h1_hbm, eexp_hbm, dinv_hbm,
                opart_hbm,
                srcv, dstv, hrows, eev, dvv, osh):
    cid, sid, wid, n_i = _wid_and_niter(NCHUNKS)
    hrowsf = hrows.reshape(CHUNK * D1)
    eevf = eev.reshape(CHUNK * NH1)
    dvvf = dvv.reshape(CHUNK * NH1)
    zero = jnp.zeros((16,), jnp.float32)

    def zb(j, _):
        hrowsf[pl.ds(16 * j, 16)] = zero
        return 0
    lax.fori_loop(0, CHUNK * D1 // 16, zb, 0)
    pltpu.sync_copy(hrows.at[pl.ds(0, CHUNK)], osh.at[pl.ds(sid * ROWS_PT, CHUNK)])
    pltpu.sync_copy(hrows.at[pl.ds(0, ROWS_PT - CHUNK)],
                    osh.at[pl.ds(sid * ROWS_PT + CHUNK, ROWS_PT - CHUNK)])
    plsc.subcore_barrier()

    q = lax.iota(jnp.int32, 16)
    bidx = [(2 * s + lax.shift_right_logical(q, 3)) & 15 for s in range(NH1)]

    def chunk_body(i, _):
        c = wid + NWORK * i
        pltpu.sync_copy(src_hbm.at[pl.ds(NSUB * c, NSUB)], srcv)
        pltpu.sync_copy(dst_hbm.at[pl.ds(NSUB * c, NSUB)], dstv)
        for k in range(NSUB):
            pltpu.sync_copy(h1_hbm.at[srcv.at[k]], hrows.at[pl.ds(SUB * k, SUB)])
            pltpu.sync_copy(dinv_hbm.at[dstv.at[k]], dvv.at[pl.ds(SUB * k, SUB)])
        pltpu.sync_copy(eexp_hbm.at[pl.ds(CHUNK * c, CHUNK)], eev)

        def inner(j, _):
            al = eevf[pl.ds(16 * j, 16)] * dvvf[pl.ds(16 * j, 16)]
            for s in range(NH1):
                av = jnp.take(al, bidx[s], mode="promise_in_bounds")
                off = 128 * j + 16 * s
                hrowsf[pl.ds(off, 16)] = hrowsf[pl.ds(off, 16)] * av
            return 0
        lax.fori_loop(0, CHUNK // 2, inner, 0)

        for k in range(NSUB):
            pltpu.sync_copy(hrows.at[pl.ds(SUB * k, SUB)], osh.at[dstv.at[k]],
                            add=True)
        return 0
    lax.fori_loop(0, n_i, chunk_body, 0)

    plsc.subcore_barrier()
    pltpu.sync_copy(osh.at[pl.ds(sid * ROWS_PT, ROWS_PT)],
                    opart_hbm.at[cid, pl.ds(sid * ROWS_PT, ROWS_PT)])


def _sc_a2_body(src_hbm, dst_hbm, as2_hbm, ad2_hbm, g2_hbm,
                eexp2_hbm, d2part_hbm,
                srcv, dstv, as2t, ad2t, eevc, eevw, gv, dsh2):
    cid, sid, wid, n_i = _wid_and_niter(NCHUNKS)
    srcvf = srcv.reshape(CHUNK)
    dstvf = dstv.reshape(CHUNK)
    eevwf = eevw.reshape(CHUNK * 8)
    zero = jnp.zeros((16,), jnp.float32)

    def zb(j, _):
        eevwf[pl.ds(16 * j, 16)] = zero
        return 0
    lax.fori_loop(0, CHUNK * 8 // 16, zb, 0)
    pltpu.sync_copy(eevw.at[pl.ds(0, CHUNK)], dsh2.at[pl.ds(sid * ROWS_PT, CHUNK)])
    pltpu.sync_copy(eevw.at[pl.ds(0, ROWS_PT - CHUNK)],
                    dsh2.at[pl.ds(sid * ROWS_PT + CHUNK, ROWS_PT - CHUNK)])
    pltpu.sync_copy(as2_hbm, as2t)
    pltpu.sync_copy(ad2_hbm, ad2t)
    pltpu.sync_copy(g2_hbm, gv)
    plsc.subcore_barrier()

    g = gv[...]
    q = lax.iota(jnp.int32, 16)
    q8 = q * 8

    def chunk_body(i, _):
        c = wid + NWORK * i
        pltpu.sync_copy(src_hbm.at[pl.ds(NSUB * c, NSUB)], srcv)
        pltpu.sync_copy(dst_hbm.at[pl.ds(NSUB * c, NSUB)], dstv)

        def inner(j, _):
            s16 = srcvf[pl.ds(16 * j, 16)]
            d16 = dstvf[pl.ds(16 * j, 16)]
            a_s = plsc.load_gather(as2t, [s16])
            a_d = plsc.load_gather(ad2t, [d16])
            e = _leaky(a_s + a_d)
            cb = _leaky(g + a_d)
            ee = jnp.exp(e - cb)
            eevc[pl.ds(16 * j, 16)] = ee
            plsc.store_scatter(eevwf, [128 * j + q8], ee)
            return 0
        lax.fori_loop(0, CHUNK // 16, inner, 0)

        for k in range(NSUB):
            pltpu.sync_copy(eevw.at[pl.ds(SUB * k, SUB)], dsh2.at[dstv.at[k]],
                            add=True)
        pltpu.sync_copy(eevc, eexp2_hbm.at[pl.ds(CHUNK * c, CHUNK)])
        return 0
    lax.fori_loop(0, n_i, chunk_body, 0)

    plsc.subcore_barrier()
    pltpu.sync_copy(dsh2.at[pl.ds(sid * ROWS_PT, ROWS_PT)],
                    d2part_hbm.at[cid, pl.ds(sid * ROWS_PT, ROWS_PT)])


def _sc_b2_body(src_hbm, dst_hbm, h2_hbm, eexp2_hbm, dinv2_hbm,
                o2part_hbm,
                srcv, dstv, hrows, eevc, dv2t, osh2):
    cid, sid, wid, n_i = _wid_and_niter(NCHUNKS)
    dstvf = dstv.reshape(CHUNK)
    hrowsf = hrows.reshape(CHUNK * D2)
    zero = jnp.zeros((16,), jnp.float32)

    def zb(j, _):
        hrowsf[pl.ds(16 * j, 16)] = zero
        return 0
    lax.fori_loop(0, CHUNK * D2 // 16, zb, 0)
    pltpu.sync_copy(hrows.at[pl.ds(0, CHUNK)], osh2.at[pl.ds(sid * ROWS_PT, CHUNK)])
    pltpu.sync_copy(hrows.at[pl.ds(0, ROWS_PT - CHUNK)],
                    osh2.at[pl.ds(sid * ROWS_PT + CHUNK, ROWS_PT - CHUNK)])
    pltpu.sync_copy(dinv2_hbm, dv2t)
    plsc.subcore_barrier()

    q = lax.iota(jnp.int32, 16)
    sidx = [q * 0 + k for k in range(16)]

    def chunk_body(i, _):
        c = wid + NWORK * i
        pltpu.sync_copy(src_hbm.at[pl.ds(NSUB * c, NSUB)], srcv)
        pltpu.sync_copy(dst_hbm.at[pl.ds(NSUB * c, NSUB)], dstv)
        for k in range(NSUB):
            pltpu.sync_copy(h2_hbm.at[srcv.at[k]], hrows.at[pl.ds(SUB * k, SUB)])
        pltpu.sync_copy(eexp2_hbm.at[pl.ds(CHUNK * c, CHUNK)], eevc)

        def inner(j, _):
            d16 = dstvf[pl.ds(16 * j, 16)]
            al = eevc[pl.ds(16 * j, 16)] * plsc.load_gather(dv2t, [d16])
            for k in range(16):
                av = jnp.take(al, sidx[k], mode="promise_in_bounds")
                base = D2 * 16 * j + D2 * k
                for s in range(D2 // 16):
                    off = base + 16 * s
                    hrowsf[pl.ds(off, 16)] = hrowsf[pl.ds(off, 16)] * av
            return 0
        lax.fori_loop(0, CHUNK // 16, inner, 0)

        for k in range(NSUB):
            pltpu.sync_copy(hrows.at[pl.ds(SUB * k, SUB)], osh2.at[dstv.at[k]],
                            add=True)
        return 0
    lax.fori_loop(0, n_i, chunk_body, 0)

    plsc.subcore_barrier()
    pltpu.sync_copy(osh2.at[pl.ds(sid * ROWS_PT, ROWS_PT)],
                    o2part_hbm.at[cid, pl.ds(sid * ROWS_PT, ROWS_PT)])


def kernel(x, edge_index, W1, a1_src, a1_dst, W2, a2_src, a2_dst):
    f32 = jnp.float32
    src = edge_index[0].astype(jnp.int32)
    dst = edge_index[1].astype(jnp.int32)
    src2 = src.reshape(EB // SUB, SUB)
    dst2 = dst.reshape(EB // SUB, SUB)

    eye = jnp.eye(NH1, dtype=f32)
    ams = (a1_src[:, :, None] * eye[:, None, :]).reshape(D1, NH1)
    amd = (a1_dst[:, :, None] * eye[:, None, :]).reshape(D1, NH1)

    h1, as1, ad1, g8 = pl.pallas_call(
        _tc1_body,
        out_shape=(
            jax.ShapeDtypeStruct((NB, D1), f32),
            jax.ShapeDtypeStruct((NB, NH1), f32),
            jax.ShapeDtypeStruct((NB, NH1), f32),
            jax.ShapeDtypeStruct((1, NH1), f32),
        ),
    )(x, W1, ams, amd)
    g16 = jnp.tile(g8[0], 2)

    mesh = plsc.VectorSubcoreMesh(core_axis_name="c", subcore_axis_name="s")

    eexp1, dpart = pl.kernel(
        _sc_a1_body,
        out_type=(
            jax.ShapeDtypeStruct((EB, NH1), f32),
            jax.ShapeDtypeStruct((2, NB, NH1), f32),
        ),
        mesh=mesh,
        scratch_types=[
            pltpu.VMEM((NSUB, SUB), jnp.int32),
            pltpu.VMEM((NSUB, SUB), jnp.int32),
            pltpu.VMEM((CHUNK, NH1), f32),
            pltpu.VMEM((CHUNK, NH1), f32),
            pltpu.VMEM((CHUNK, NH1), f32),
            pltpu.VMEM((16,), f32),
            pltpu.VMEM_SHARED((NB, NH1), f32),
        ],
    )(src2, dst2, as1, ad1, g16)

    dinv1 = pl.pallas_call(
        _tc2_body, out_shape=jax.ShapeDtypeStruct((NB, NH1), f32),
    )(dpart)

    opart = pl.kernel(
        _sc_b1_body,
        out_type=jax.ShapeDtypeStruct((2, NB, D1), f32),
        mesh=mesh,
        scratch_types=[
            pltpu.VMEM((NSUB, SUB), jnp.int32),
            pltpu.VMEM((NSUB, SUB), jnp.int32),
            pltpu.VMEM((CHUNK, D1), f32),
            pltpu.VMEM((CHUNK, NH1), f32),
            pltpu.VMEM((CHUNK, NH1), f32),
            pltpu.VMEM_SHARED((NB, D1), f32),
        ],
    )(src2, dst2, h1, eexp1, dinv1)

    w2p = jnp.pad(W2, ((0, 0), (0, D2 - NC2)))
    as2v = jnp.pad(a2_src[0], (0, D2 - NC2)).reshape(D2, 1)
    ad2v = jnp.pad(a2_dst[0], (0, D2 - NC2)).reshape(D2, 1)

    h2, s2, d2, g2 = pl.pallas_call(
        _tc3_body,
        out_shape=(
            jax.ShapeDtypeStruct((NB, D2), f32),
            jax.ShapeDtypeStruct((NB, 1), f32),
            jax.ShapeDtypeStruct((NB, 1), f32),
            jax.ShapeDtypeStruct((1, 1), f32),
        ),
    )(opart, w2p, as2v, ad2v)
    g2_16 = jnp.broadcast_to(g2[0, 0], (16,))

    eexp2, d2part = pl.kernel(
        _sc_a2_body,
        out_type=(
            jax.ShapeDtypeStruct((EB,), f32),
            jax.ShapeDtypeStruct((2, NB, 8), f32),
        ),
        mesh=mesh,
        scratch_types=[
            pltpu.VMEM((NSUB, SUB), jnp.int32),
            pltpu.VMEM((NSUB, SUB), jnp.int32),
            pltpu.VMEM((NB,), f32),
            pltpu.VMEM((NB,), f32),
            pltpu.VMEM((CHUNK,), f32),
            pltpu.VMEM((CHUNK, 8), f32),
            pltpu.VMEM((16,), f32),
            pltpu.VMEM_SHARED((NB, 8), f32),
        ],
    )(src2, dst2, s2.reshape(NB), d2.reshape(NB), g2_16)

    dinv2 = pl.pallas_call(
        _tc2_body, out_shape=jax.ShapeDtypeStruct((NB, 8), f32),
    )(d2part)

    o2part = pl.kernel(
        _sc_b2_body,
        out_type=jax.ShapeDtypeStruct((2, NB, D2), f32),
        mesh=mesh,
        scratch_types=[
            pltpu.VMEM((NSUB, SUB), jnp.int32),
            pltpu.VMEM((NSUB, SUB), jnp.int32),
            pltpu.VMEM((CHUNK, D2), f32),
            pltpu.VMEM((CHUNK,), f32),
            pltpu.VMEM((NB,), f32),
            pltpu.VMEM_SHARED((NB, D2), f32),
        ],
    )(src2, dst2, h2, eexp2, dinv2[:, 0])

    out = pl.pallas_call(
        _tc5_body, out_shape=jax.ShapeDtypeStruct((NB, NC2), f32),
    )(o2part)
    return out


# speedup vs baseline: 39.2828x; 39.2828x over previous
import jax
import jax.numpy as jnp
from jax import lax
from jax.experimental import pallas as pl
from jax.experimental.pallas import tpu as pltpu
from jax.experimental.pallas import tpu_sc as plsc

NB = 10000
EB = 320000
NH1 = 8
D1 = 64
NC2 = 40
D2 = 48

CHUNK = 512
SUB = 128
NSUB = CHUNK // SUB
NCHUNKS = EB // CHUNK
NWORK = 32
NBP = 10240
ROWS_PT = NBP // 16


def _leaky(x):
    return jnp.maximum(x, 0.2 * x)


_GDN = lax.GatherDimensionNumbers(
    offset_dims=(), collapsed_slice_dims=(0,), start_index_map=(0,))


def _vgather(v, idx16):
    return lax.gather(v, idx16[:, None], _GDN, (1,),
                      mode=lax.GatherScatterMode.PROMISE_IN_BOUNDS)


def _tc1_body(x_ref, w1_ref, ams_ref, amd_ref,
              h0_ref, h1_ref, h2_ref, h3_ref, as_ref, ad_ref, g_ref):
    h = jnp.dot(x_ref[...], w1_ref[...], preferred_element_type=jnp.float32)
    h0_ref[...] = h[:, 0:16]
    h1_ref[...] = h[:, 16:32]
    h2_ref[...] = h[:, 32:48]
    h3_ref[...] = h[:, 48:64]
    as_ref[...] = jnp.dot(h, ams_ref[...], preferred_element_type=jnp.float32)
    ad_ref[...] = jnp.dot(h, amd_ref[...], preferred_element_type=jnp.float32)
    g_ref[...] = jnp.max(as_ref[...], axis=0, keepdims=True)


def _tc2_body(dp_ref, dinv_ref):
    dinv_ref[...] = 1.0 / (dp_ref[0] + dp_ref[1] + 1e-16)


def _tc3_body(p0_ref, p1_ref, p2_ref, p3_ref, w2_ref, as2_ref, ad2_ref,
              h0_ref, h1_ref, h2_ref, s2_ref, d2_ref, g2_ref):
    i = pl.program_id(0)
    o = jnp.concatenate(
        [p0_ref[0] + p0_ref[1], p1_ref[0] + p1_ref[1],
         p2_ref[0] + p2_ref[1], p3_ref[0] + p3_ref[1]], axis=1)
    hact = jnp.where(o > 0, o, jnp.exp(o) - 1.0)
    h2 = jnp.dot(hact, w2_ref[...], preferred_element_type=jnp.float32)
    h0_ref[...] = h2[:, 0:16]
    h1_ref[...] = h2[:, 16:32]
    h2_ref[...] = h2[:, 32:48]
    s2 = jnp.dot(h2, as2_ref[...], preferred_element_type=jnp.float32)
    d2 = jnp.dot(h2, ad2_ref[...], preferred_element_type=jnp.float32)
    s2_ref[...] = s2
    d2_ref[...] = d2
    m = jnp.max(s2, axis=0, keepdims=True)

    @pl.when(i == 0)
    def _():
        g2_ref[...] = m

    @pl.when(i > 0)
    def _():
        g2_ref[...] = jnp.maximum(g2_ref[...], m)


def _tc5_body(p0_ref, p1_ref, p2_ref, out_ref):
    o = jnp.concatenate(
        [p0_ref[0] + p0_ref[1], p1_ref[0] + p1_ref[1],
         p2_ref[0] + p2_ref[1]], axis=1)
    t = o[:, :NC2]
    m = jnp.max(t, axis=1, keepdims=True)
    lse = jnp.log(jnp.sum(jnp.exp(t - m), axis=1, keepdims=True))
    out_ref[...] = t - m - lse


def _wid_and_niter():
    cid = lax.axis_index("c")
    sid = lax.axis_index("s")
    wid = sid * 2 + cid
    n_i = (NCHUNKS - wid + NWORK - 1) // NWORK
    return cid, sid, wid, n_i


def _zero_shared_slice(zbuf, shared, sid):
    pltpu.sync_copy(zbuf.at[pl.ds(0, CHUNK)],
                    shared.at[pl.ds(sid * ROWS_PT, CHUNK)])
    pltpu.sync_copy(zbuf.at[pl.ds(0, ROWS_PT - CHUNK)],
                    shared.at[pl.ds(sid * ROWS_PT + CHUNK, ROWS_PT - CHUNK)])


def _idx16(ref, j):
    return ref[j // (SUB // 16), pl.ds(16 * (j % (SUB // 16)), 16)]


def _sc_a1_body(src_hbm, dst_hbm, as_hbm, ad_hbm, g_hbm,
                eexp_hbm, dpart_hbm,
                srcv, dstv, asv, adv, eev, gv, dsh):
    cid, sid, wid, n_i = _wid_and_niter()
    zero = jnp.zeros((16,), jnp.float32)

    def zb(j, _):
        eev[j] = zero
        return 0
    lax.fori_loop(0, CHUNK, zb, 0)
    _zero_shared_slice(eev, dsh, sid)
    pltpu.sync_copy(g_hbm, gv)
    plsc.subcore_barrier()

    g = gv[...]

    def chunk_body(i, _):
        c = wid + NWORK * i
        pltpu.sync_copy(src_hbm.at[c], srcv)
        pltpu.sync_copy(dst_hbm.at[c], dstv)
        for k in range(NSUB):
            pltpu.sync_copy(as_hbm.at[srcv.at[k]], asv.at[pl.ds(SUB * k, SUB)])
            pltpu.sync_copy(ad_hbm.at[dstv.at[k]], adv.at[pl.ds(SUB * k, SUB)])

        def inner(r, _):
            s = asv[r]
            a = adv[r]
            e = _leaky(s + a)
            cb = _leaky(g + a)
            eev[r] = jnp.exp(e - cb)
            return 0
        lax.fori_loop(0, CHUNK, inner, 0)

        for k in range(NSUB):
            pltpu.sync_copy(eev.at[pl.ds(SUB * k, SUB)], dsh.at[dstv.at[k]],
                            add=True)
        pltpu.sync_copy(eev, eexp_hbm.at[pl.ds(CHUNK * c, CHUNK)])
        return 0
    lax.fori_loop(0, n_i, chunk_body, 0)

    plsc.subcore_barrier()
    pltpu.sync_copy(dsh.at[pl.ds(sid * ROWS_PT, ROWS_PT)],
                    dpart_hbm.at[cid, pl.ds(sid * ROWS_PT, ROWS_PT)])


def _sc_b1_body(src_hbm, dst_hbm, t0_hbm, t1_hbm, t2_hbm, t3_hbm,
                eexp_hbm, dinv_hbm,
                p0_hbm, p1_hbm, p2_hbm, p3_hbm,
                srcv, dstv, hr0, hr1, hr2, hr3, eev, dvv,
                osh0, osh1, osh2, osh3):
    cid, sid, wid, n_i = _wid_and_niter()
    hrs = (hr0, hr1, hr2, hr3)
    oshs = (osh0, osh1, osh2, osh3)
    tabs = (t0_hbm, t1_hbm, t2_hbm, t3_hbm)
    parts = (p0_hbm, p1_hbm, p2_hbm, p3_hbm)
    zero = jnp.zeros((16,), jnp.float32)

    def zb(j, _):
        hr0[j] = zero
        return 0
    lax.fori_loop(0, CHUNK, zb, 0)
    for t in range(4):
        _zero_shared_slice(hr0, oshs[t], sid)
    plsc.subcore_barrier()

    q = lax.iota(jnp.int32, 16)
    bidx = [2 * t + lax.shift_right_logical(q, 3) for t in range(4)]

    def chunk_body(i, _):
        c = wid + NWORK * i
        pltpu.sync_copy(src_hbm.at[c], srcv)
        pltpu.sync_copy(dst_hbm.at[c], dstv)
        for k in range(NSUB):
            for t in range(4):
                pltpu.sync_copy(tabs[t].at[srcv.at[k]],
                                hrs[t].at[pl.ds(SUB * k, SUB)])
            pltpu.sync_copy(dinv_hbm.at[dstv.at[k]], dvv.at[pl.ds(SUB * k, SUB)])
        pltpu.sync_copy(eexp_hbm.at[pl.ds(CHUNK * c, CHUNK)], eev)

        def inner(r, _):
            al = eev[r] * dvv[r]
            for t in range(4):
                av = _vgather(al, bidx[t])
                hrs[t][r] = hrs[t][r] * av
            return 0
        lax.fori_loop(0, CHUNK, inner, 0)

        for k in range(NSUB):
            for t in range(4):
                pltpu.sync_copy(hrs[t].at[pl.ds(SUB * k, SUB)],
                                oshs[t].at[dstv.at[k]], add=True)
        return 0
    lax.fori_loop(0, n_i, chunk_body, 0)

    plsc.subcore_barrier()
    for t in range(4):
        pltpu.sync_copy(oshs[t].at[pl.ds(sid * ROWS_PT, ROWS_PT)],
                        parts[t].at[cid, pl.ds(sid * ROWS_PT, ROWS_PT)])


def _sc_a2_body(src_hbm, dst_hbm, as2_hbm, ad2_hbm, g2_hbm,
                eexp2_hbm, d2part_hbm,
                srcv, dstv, as2t, ad2t, eevc, eevw, gv, dsh2):
    cid, sid, wid, n_i = _wid_and_niter()
    zero = jnp.zeros((16,), jnp.float32)

    def zb(j, _):
        eevw[j] = zero
        return 0
    lax.fori_loop(0, CHUNK, zb, 0)
    _zero_shared_slice(eevw, dsh2, sid)
    pltpu.sync_copy(as2_hbm, as2t)
    pltpu.sync_copy(ad2_hbm, ad2t)
    pltpu.sync_copy(g2_hbm, gv)
    plsc.subcore_barrier()

    g = gv[...]
    q = lax.iota(jnp.int32, 16)
    zcol = q * 0

    def chunk_body(i, _):
        c = wid + NWORK * i
        pltpu.sync_copy(src_hbm.at[c], srcv)
        pltpu.sync_copy(dst_hbm.at[c], dstv)

        def inner(j, _):
            s16 = _idx16(srcv, j)
            d16 = _idx16(dstv, j)
            a_s = plsc.load_gather(as2t, [s16])
            a_d = plsc.load_gather(ad2t, [d16])
            e = _leaky(a_s + a_d)
            cb = _leaky(g + a_d)
            ee = jnp.exp(e - cb)
            eevc[j] = ee
            plsc.store_scatter(eevw, [16 * j + q, zcol], ee)
            return 0
        lax.fori_loop(0, CHUNK // 16, inner, 0)

        for k in range(NSUB):
            pltpu.sync_copy(eevw.at[pl.ds(SUB * k, SUB)], dsh2.at[dstv.at[k]],
                            add=True)
        pltpu.sync_copy(eevc, eexp2_hbm.at[pl.ds(CHUNK // 16 * c, CHUNK // 16)])
        return 0
    lax.fori_loop(0, n_i, chunk_body, 0)

    plsc.subcore_barrier()
    pltpu.sync_copy(dsh2.at[pl.ds(sid * ROWS_PT, ROWS_PT)],
                    d2part_hbm.at[cid, pl.ds(sid * ROWS_PT, ROWS_PT)])


def _sc_b2_body(src_hbm, dst_hbm, t0_hbm, t1_hbm, t2_hbm,
                eexp2_hbm, dinv2_hbm,
                p0_hbm, p1_hbm, p2_hbm,
                srcv, dstv, hr0, hr1, hr2, eevc, dv2t,
                osh0, osh1, osh2):
    cid, sid, wid, n_i = _wid_and_niter()
    hrs = (hr0, hr1, hr2)
    oshs = (osh0, osh1, osh2)
    tabs = (t0_hbm, t1_hbm, t2_hbm)
    parts = (p0_hbm, p1_hbm, p2_hbm)
    zero = jnp.zeros((16,), jnp.float32)

    def zb(j, _):
        hr0[j] = zero
        return 0
    lax.fori_loop(0, CHUNK, zb, 0)
    for t in range(3):
        _zero_shared_slice(hr0, oshs[t], sid)
    pltpu.sync_copy(dinv2_hbm, dv2t)
    plsc.subcore_barrier()

    q = lax.iota(jnp.int32, 16)
    sidx = [q * 0 + k for k in range(16)]

    def chunk_body(i, _):
        c = wid + NWORK * i
        pltpu.sync_copy(src_hbm.at[c], srcv)
        pltpu.sync_copy(dst_hbm.at[c], dstv)
        for k in range(NSUB):
            for t in range(3):
                pltpu.sync_copy(tabs[t].at[srcv.at[k]],
                                hrs[t].at[pl.ds(SUB * k, SUB)])
        pltpu.sync_copy(eexp2_hbm.at[pl.ds(CHUNK // 16 * c, CHUNK // 16)], eevc)

        def inner(j, _):
            d16 = _idx16(dstv, j)
            al = eevc[j] * plsc.load_gather(dv2t, [d16])
            for k in range(16):
                av = _vgather(al, sidx[k])
                r = 16 * j + k
                for t in range(3):
                    hrs[t][r] = hrs[t][r] * av
            return 0
        lax.fori_loop(0, CHUNK // 16, inner, 0)

        for k in range(NSUB):
            for t in range(3):
                pltpu.sync_copy(hrs[t].at[pl.ds(SUB * k, SUB)],
                                oshs[t].at[dstv.at[k]], add=True)
        return 0
    lax.fori_loop(0, n_i, chunk_body, 0)

    plsc.subcore_barrier()
    for t in range(3):
        pltpu.sync_copy(oshs[t].at[pl.ds(sid * ROWS_PT, ROWS_PT)],
                        parts[t].at[cid, pl.ds(sid * ROWS_PT, ROWS_PT)])


def kernel(x, edge_index, W1, a1_src, a1_dst, W2, a2_src, a2_dst):
    f32 = jnp.float32
    i32 = jnp.int32
    src = edge_index[0].astype(i32)
    dst = edge_index[1].astype(i32)
    src2 = src.reshape(NCHUNKS, NSUB, SUB)
    dst2 = dst.reshape(NCHUNKS, NSUB, SUB)

    eye = jnp.eye(NH1, dtype=f32)
    ams = jnp.pad((a1_src[:, :, None] * eye[:, None, :]).reshape(D1, NH1),
                  ((0, 0), (0, 8)))
    amd = jnp.pad((a1_dst[:, :, None] * eye[:, None, :]).reshape(D1, NH1),
                  ((0, 0), (0, 8)))

    h1c0, h1c1, h1c2, h1c3, as16, ad16, g8 = pl.pallas_call(
        _tc1_body,
        out_shape=(
            jax.ShapeDtypeStruct((NB, 16), f32),
            jax.ShapeDtypeStruct((NB, 16), f32),
            jax.ShapeDtypeStruct((NB, 16), f32),
            jax.ShapeDtypeStruct((NB, 16), f32),
            jax.ShapeDtypeStruct((NB, 16), f32),
            jax.ShapeDtypeStruct((NB, 16), f32),
            jax.ShapeDtypeStruct((1, 16), f32),
        ),
    )(x, W1, ams, amd)
    g16 = jnp.where(jnp.arange(16) < NH1, g8[0], 40.0)

    mesh = plsc.VectorSubcoreMesh(core_axis_name="c", subcore_axis_name="s")
    idx_t = pltpu.VMEM((NSUB, SUB), i32)
    sc_params = pltpu.CompilerParams(use_tc_tiling_on_sc=False, needs_layout_passes=False)

    eexp1, dpart = pl.kernel(
        _sc_a1_body,
        out_type=(
            jax.ShapeDtypeStruct((EB, 16), f32),
            jax.ShapeDtypeStruct((2, NBP, 16), f32),
        ),
        mesh=mesh,
        compiler_params=sc_params,
        scratch_types=[
            idx_t, idx_t,
            pltpu.VMEM((CHUNK, 16), f32),
            pltpu.VMEM((CHUNK, 16), f32),
            pltpu.VMEM((CHUNK, 16), f32),
            pltpu.VMEM((16,), f32),
            pltpu.VMEM_SHARED((NBP, 16), f32),
        ],
    )(src2, dst2, as16, ad16, g16)

    dinv1 = pl.pallas_call(
        _tc2_body, out_shape=jax.ShapeDtypeStruct((NBP, 16), f32),
    )(dpart)

    op0, op1, op2, op3 = pl.kernel(
        _sc_b1_body,
        out_type=tuple(jax.ShapeDtypeStruct((2, NBP, 16), f32) for _ in range(4)),
        mesh=mesh,
        compiler_params=sc_params,
        scratch_types=[
            idx_t, idx_t,
            pltpu.VMEM((CHUNK, 16), f32),
            pltpu.VMEM((CHUNK, 16), f32),
            pltpu.VMEM((CHUNK, 16), f32),
            pltpu.VMEM((CHUNK, 16), f32),
            pltpu.VMEM((CHUNK, 16), f32),
            pltpu.VMEM((CHUNK, 16), f32),
            pltpu.VMEM_SHARED((NBP, 16), f32),
            pltpu.VMEM_SHARED((NBP, 16), f32),
            pltpu.VMEM_SHARED((NBP, 16), f32),
            pltpu.VMEM_SHARED((NBP, 16), f32),
        ],
    )(src2, dst2, h1c0, h1c1, h1c2, h1c3, eexp1, dinv1)

    w2p = jnp.pad(W2, ((0, 0), (0, D2 - NC2)))
    as2v = jnp.pad(a2_src[0], (0, D2 - NC2)).reshape(D2, 1)
    ad2v = jnp.pad(a2_dst[0], (0, D2 - NC2)).reshape(D2, 1)

    b3 = 1024
    pspec = pl.BlockSpec((2, b3, 16), lambda i: (0, i, 0))
    h2c0, h2c1, h2c2, s2, d2, g2 = pl.pallas_call(
        _tc3_body,
        grid=(NBP // b3,),
        in_specs=[pspec, pspec, pspec, pspec,
                  pl.BlockSpec((D1, D2), lambda i: (0, 0)),
                  pl.BlockSpec((D2, 1), lambda i: (0, 0)),
                  pl.BlockSpec((D2, 1), lambda i: (0, 0))],
        out_specs=(
            pl.BlockSpec((b3, 16), lambda i: (i, 0)),
            pl.BlockSpec((b3, 16), lambda i: (i, 0)),
            pl.BlockSpec((b3, 16), lambda i: (i, 0)),
            pl.BlockSpec((b3, 1), lambda i: (i, 0)),
            pl.BlockSpec((b3, 1), lambda i: (i, 0)),
            pl.BlockSpec((1, 1), lambda i: (0, 0)),
        ),
        out_shape=(
            jax.ShapeDtypeStruct((NBP, 16), f32),
            jax.ShapeDtypeStruct((NBP, 16), f32),
            jax.ShapeDtypeStruct((NBP, 16), f32),
            jax.ShapeDtypeStruct((NBP, 1), f32),
            jax.ShapeDtypeStruct((NBP, 1), f32),
            jax.ShapeDtypeStruct((1, 1), f32),
        ),
    )(op0, op1, op2, op3, w2p, as2v, ad2v)
    g2_16 = jnp.broadcast_to(g2[0, 0], (16,))

    eexp2, d2part = pl.kernel(
        _sc_a2_body,
        out_type=(
            jax.ShapeDtypeStruct((EB // 16, 16), f32),
            jax.ShapeDtypeStruct((2, NBP, 16), f32),
        ),
        mesh=mesh,
        compiler_params=sc_params,
        scratch_types=[
            idx_t, idx_t,
            pltpu.VMEM((NBP,), f32),
            pltpu.VMEM((NBP,), f32),
            pltpu.VMEM((CHUNK // 16, 16), f32),
            pltpu.VMEM((CHUNK, 16), f32),
            pltpu.VMEM((16,), f32),
            pltpu.VMEM_SHARED((NBP, 16), f32),
        ],
    )(src2, dst2, s2.reshape(NBP), d2.reshape(NBP), g2_16)

    dinv2 = pl.pallas_call(
        _tc2_body, out_shape=jax.ShapeDtypeStruct((NBP, 16), f32),
    )(d2part)

    o2p0, o2p1, o2p2 = pl.kernel(
        _sc_b2_body,
        out_type=tuple(jax.ShapeDtypeStruct((2, NBP, 16), f32) for _ in range(3)),
        mesh=mesh,
        compiler_params=sc_params,
        scratch_types=[
            idx_t, idx_t,
            pltpu.VMEM((CHUNK, 16), f32),
            pltpu.VMEM((CHUNK, 16), f32),
            pltpu.VMEM((CHUNK, 16), f32),
            pltpu.VMEM((CHUNK // 16, 16), f32),
            pltpu.VMEM((NBP,), f32),
            pltpu.VMEM_SHARED((NBP, 16), f32),
            pltpu.VMEM_SHARED((NBP, 16), f32),
            pltpu.VMEM_SHARED((NBP, 16), f32),
        ],
    )(src2, dst2, h2c0, h2c1, h2c2, eexp2, dinv2[:, 0])

    b5 = 1000
    pspec5 = pl.BlockSpec((2, b5, 16), lambda i: (0, i, 0))
    out = pl.pallas_call(
        _tc5_body,
        grid=(NB // b5,),
        in_specs=[pspec5, pspec5, pspec5],
        out_specs=pl.BlockSpec((b5, NC2), lambda i: (i, 0)),
        out_shape=jax.ShapeDtypeStruct((NB, NC2), f32),
    )(o2p0, o2p1, o2p2)
    return out


# speedup vs baseline: 64.3804x; 1.6389x over previous
import jax
import jax.numpy as jnp
from jax import lax
from jax.experimental import pallas as pl
from jax.experimental.pallas import tpu as pltpu
from jax.experimental.pallas import tpu_sc as plsc

NB = 10000
EB = 320000
NH1 = 8
D1 = 64
NC2 = 40
D2 = 48

CHUNK = 512
SUB = 128
NSUB = CHUNK // SUB
NCHUNKS = EB // CHUNK
NWORK = 32
NBP = 10240
ROWS_PT = NBP // 16


def _leaky(x):
    return jnp.maximum(x, 0.2 * x)


_GDN = lax.GatherDimensionNumbers(
    offset_dims=(), collapsed_slice_dims=(0,), start_index_map=(0,))


def _vgather(v, idx16):
    return lax.gather(v, idx16[:, None], _GDN, (1,),
                      mode=lax.GatherScatterMode.PROMISE_IN_BOUNDS)


def _tc1_body(x_ref, w1_ref, ams_ref, amd_ref,
              h0_ref, h1_ref, h2_ref, h3_ref, as_ref, ad_ref, g_ref):
    h = jnp.dot(x_ref[...], w1_ref[...], preferred_element_type=jnp.float32)
    h0_ref[...] = h[:, 0:16]
    h1_ref[...] = h[:, 16:32]
    h2_ref[...] = h[:, 32:48]
    h3_ref[...] = h[:, 48:64]
    as_ref[...] = jnp.dot(h, ams_ref[...], preferred_element_type=jnp.float32)
    ad_ref[...] = jnp.dot(h, amd_ref[...], preferred_element_type=jnp.float32)
    g_ref[...] = jnp.max(as_ref[...], axis=0, keepdims=True)


def _tc2_body(dp_ref, dinv_ref):
    dinv_ref[...] = 1.0 / (dp_ref[0] + dp_ref[1] + 1e-16)


def _tc3_body(p0_ref, p1_ref, p2_ref, p3_ref, w2_ref, as2_ref, ad2_ref,
              h0_ref, h1_ref, h2_ref, s2_ref, d2_ref, g2_ref):
    i = pl.program_id(0)
    o = jnp.concatenate(
        [p0_ref[0] + p0_ref[1], p1_ref[0] + p1_ref[1],
         p2_ref[0] + p2_ref[1], p3_ref[0] + p3_ref[1]], axis=1)
    hact = jnp.where(o > 0, o, jnp.exp(o) - 1.0)
    h2 = jnp.dot(hact, w2_ref[...], preferred_element_type=jnp.float32)
    h0_ref[...] = h2[:, 0:16]
    h1_ref[...] = h2[:, 16:32]
    h2_ref[...] = h2[:, 32:48]
    s2 = jnp.dot(h2, as2_ref[...], preferred_element_type=jnp.float32)
    d2 = jnp.dot(h2, ad2_ref[...], preferred_element_type=jnp.float32)
    s2_ref[...] = s2
    d2_ref[...] = d2
    m = jnp.max(s2, axis=0, keepdims=True)

    @pl.when(i == 0)
    def _():
        g2_ref[...] = m

    @pl.when(i > 0)
    def _():
        g2_ref[...] = jnp.maximum(g2_ref[...], m)


def _tc5_body(p0_ref, p1_ref, p2_ref, out_ref):
    o = jnp.concatenate(
        [p0_ref[0] + p0_ref[1], p1_ref[0] + p1_ref[1],
         p2_ref[0] + p2_ref[1]], axis=1)
    t = o[:, :NC2]
    m = jnp.max(t, axis=1, keepdims=True)
    lse = jnp.log(jnp.sum(jnp.exp(t - m), axis=1, keepdims=True))
    out_ref[...] = t - m - lse


def _wid_and_niter():
    cid = lax.axis_index("c")
    sid = lax.axis_index("s")
    wid = sid * 2 + cid
    n_i = (NCHUNKS - wid + NWORK - 1) // NWORK
    return cid, sid, wid, n_i


def _zero_shared_slice(zbuf, shared, sid):
    pltpu.sync_copy(zbuf.at[pl.ds(0, CHUNK)],
                    shared.at[pl.ds(sid * ROWS_PT, CHUNK)])
    pltpu.sync_copy(zbuf.at[pl.ds(0, ROWS_PT - CHUNK)],
                    shared.at[pl.ds(sid * ROWS_PT + CHUNK, ROWS_PT - CHUNK)])


def _idx16(ref, j):
    return ref[j // (SUB // 16), pl.ds(16 * (j % (SUB // 16)), 16)]


def _sc_a1_body(src_hbm, dst_hbm, as_hbm, ad_hbm, g_hbm,
                eexp_hbm, dpart_hbm,
                srcv, dstv, asv, adv, eev, gv, dsh, sem):
    cid, sid, wid, n_i = _wid_and_niter()
    zero = jnp.zeros((16,), jnp.float32)

    def zb(j, _):
        eev[j] = zero
        return 0
    lax.fori_loop(0, CHUNK, zb, 0)
    _zero_shared_slice(eev, dsh, sid)
    pltpu.sync_copy(g_hbm, gv)
    plsc.subcore_barrier()

    g = gv[...]

    def chunk_body(i, _):
        c = wid + NWORK * i
        d1 = pltpu.async_copy(src_hbm.at[c], srcv, sem)
        d2 = pltpu.async_copy(dst_hbm.at[c], dstv, sem)
        d1.wait()
        d2.wait()
        ds_ = []
        for k in range(NSUB):
            ds_.append(pltpu.async_copy(
                as_hbm.at[srcv.at[k]], asv.at[pl.ds(SUB * k, SUB)], sem))
            ds_.append(pltpu.async_copy(
                ad_hbm.at[dstv.at[k]], adv.at[pl.ds(SUB * k, SUB)], sem))
        for d in ds_:
            d.wait()

        def inner(r, _):
            s = asv[r]
            a = adv[r]
            e = _leaky(s + a)
            cb = _leaky(g + a)
            eev[r] = jnp.exp(e - cb)
            return 0
        lax.fori_loop(0, CHUNK, inner, 0)

        for k in range(NSUB):
            pltpu.sync_copy(eev.at[pl.ds(SUB * k, SUB)], dsh.at[dstv.at[k]],
                            add=True)
        pltpu.sync_copy(eev, eexp_hbm.at[pl.ds(CHUNK * c, CHUNK)])
        return 0
    lax.fori_loop(0, n_i, chunk_body, 0)

    plsc.subcore_barrier()
    pltpu.sync_copy(dsh.at[pl.ds(sid * ROWS_PT, ROWS_PT)],
                    dpart_hbm.at[cid, pl.ds(sid * ROWS_PT, ROWS_PT)])


def _sc_b1_body(src_hbm, dst_hbm, t0_hbm, t1_hbm, t2_hbm, t3_hbm,
                eexp_hbm, dinv_hbm,
                p0_hbm, p1_hbm, p2_hbm, p3_hbm,
                srcv, dstv, hr0, hr1, hr2, hr3, eev, dvv,
                osh0, osh1, osh2, osh3, sem):
    cid, sid, wid, n_i = _wid_and_niter()
    hrs = (hr0, hr1, hr2, hr3)
    oshs = (osh0, osh1, osh2, osh3)
    tabs = (t0_hbm, t1_hbm, t2_hbm, t3_hbm)
    parts = (p0_hbm, p1_hbm, p2_hbm, p3_hbm)
    zero = jnp.zeros((16,), jnp.float32)

    def zb(j, _):
        hr0[j] = zero
        return 0
    lax.fori_loop(0, CHUNK, zb, 0)
    for t in range(4):
        _zero_shared_slice(hr0, oshs[t], sid)
    plsc.subcore_barrier()

    q = lax.iota(jnp.int32, 16)
    bidx = [2 * t + lax.shift_right_logical(q, 3) for t in range(4)]

    def chunk_body(i, _):
        c = wid + NWORK * i
        d1 = pltpu.async_copy(src_hbm.at[c], srcv, sem)
        d2 = pltpu.async_copy(dst_hbm.at[c], dstv, sem)
        d1.wait()
        d2.wait()
        ds_ = [pltpu.async_copy(eexp_hbm.at[pl.ds(CHUNK * c, CHUNK)], eev, sem)]
        for k in range(NSUB):
            for t in range(4):
                ds_.append(pltpu.async_copy(
                    tabs[t].at[srcv.at[k]], hrs[t].at[pl.ds(SUB * k, SUB)], sem))
            ds_.append(pltpu.async_copy(
                dinv_hbm.at[dstv.at[k]], dvv.at[pl.ds(SUB * k, SUB)], sem))
        for d in ds_:
            d.wait()

        def inner(r, _):
            al = eev[r] * dvv[r]
            for t in range(4):
                av = _vgather(al, bidx[t])
                hrs[t][r] = hrs[t][r] * av
            return 0
        lax.fori_loop(0, CHUNK, inner, 0)

        for k in range(NSUB):
            for t in range(4):
                pltpu.sync_copy(hrs[t].at[pl.ds(SUB * k, SUB)],
                                oshs[t].at[dstv.at[k]], add=True)
        return 0
    lax.fori_loop(0, n_i, chunk_body, 0)

    plsc.subcore_barrier()
    for t in range(4):
        pltpu.sync_copy(oshs[t].at[pl.ds(sid * ROWS_PT, ROWS_PT)],
                        parts[t].at[cid, pl.ds(sid * ROWS_PT, ROWS_PT)])


def _sc_a2_body(src_hbm, dst_hbm, as2_hbm, ad2_hbm, g2_hbm,
                eexp2_hbm, d2part_hbm,
                srcv, dstv, as2t, ad2t, eevc, eevw, gv, dsh2, sem):
    cid, sid, wid, n_i = _wid_and_niter()
    zero = jnp.zeros((16,), jnp.float32)

    def zb(j, _):
        eevw[j] = zero
        return 0
    lax.fori_loop(0, CHUNK, zb, 0)
    _zero_shared_slice(eevw, dsh2, sid)
    pltpu.sync_copy(as2_hbm, as2t)
    pltpu.sync_copy(ad2_hbm, ad2t)
    pltpu.sync_copy(g2_hbm, gv)
    plsc.subcore_barrier()

    g = gv[...]
    q = lax.iota(jnp.int32, 16)
    zcol = q * 0

    def chunk_body(i, _):
        c = wid + NWORK * i
        d1 = pltpu.async_copy(src_hbm.at[c], srcv, sem)
        d2 = pltpu.async_copy(dst_hbm.at[c], dstv, sem)
        d1.wait()
        d2.wait()

        def inner(j, _):
            s16 = _idx16(srcv, j)
            d16 = _idx16(dstv, j)
            a_s = plsc.load_gather(as2t, [s16])
            a_d = plsc.load_gather(ad2t, [d16])
            e = _leaky(a_s + a_d)
            cb = _leaky(g + a_d)
            ee = jnp.exp(e - cb)
            eevc[j] = ee
            plsc.store_scatter(eevw, [16 * j + q, zcol], ee)
            return 0
        lax.fori_loop(0, CHUNK // 16, inner, 0)

        for k in range(NSUB):
            pltpu.sync_copy(eevw.at[pl.ds(SUB * k, SUB)], dsh2.at[dstv.at[k]],
                            add=True)
        pltpu.sync_copy(eevc, eexp2_hbm.at[pl.ds(CHUNK // 16 * c, CHUNK // 16)])
        return 0
    lax.fori_loop(0, n_i, chunk_body, 0)

    plsc.subcore_barrier()
    pltpu.sync_copy(dsh2.at[pl.ds(sid * ROWS_PT, ROWS_PT)],
                    d2part_hbm.at[cid, pl.ds(sid * ROWS_PT, ROWS_PT)])


def _sc_b2_body(src_hbm, dst_hbm, t0_hbm, t1_hbm, t2_hbm,
                eexp2_hbm, dinv2_hbm,
                p0_hbm, p1_hbm, p2_hbm,
                srcv, dstv, hr0, hr1, hr2, eevc, dv2t,
                osh0, osh1, osh2, sem):
    cid, sid, wid, n_i = _wid_and_niter()
    hrs = (hr0, hr1, hr2)
    oshs = (osh0, osh1, osh2)
    tabs = (t0_hbm, t1_hbm, t2_hbm)
    parts = (p0_hbm, p1_hbm, p2_hbm)
    zero = jnp.zeros((16,), jnp.float32)

    def zb(j, _):
        hr0[j] = zero
        return 0
    lax.fori_loop(0, CHUNK, zb, 0)
    for t in range(3):
        _zero_shared_slice(hr0, oshs[t], sid)
    pltpu.sync_copy(dinv2_hbm, dv2t)
    plsc.subcore_barrier()

    q = lax.iota(jnp.int32, 16)
    sidx = [q * 0 + k for k in range(16)]

    def chunk_body(i, _):
        c = wid + NWORK * i
        d1 = pltpu.async_copy(src_hbm.at[c], srcv, sem)
        d2 = pltpu.async_copy(dst_hbm.at[c], dstv, sem)
        d1.wait()
        d2.wait()
        ds_ = [pltpu.async_copy(
            eexp2_hbm.at[pl.ds(CHUNK // 16 * c, CHUNK // 16)], eevc, sem)]
        for k in range(NSUB):
            for t in range(3):
                ds_.append(pltpu.async_copy(
                    tabs[t].at[srcv.at[k]], hrs[t].at[pl.ds(SUB * k, SUB)], sem))
        for d in ds_:
            d.wait()

        def inner(j, _):
            d16 = _idx16(dstv, j)
            al = eevc[j] * plsc.load_gather(dv2t, [d16])
            for k in range(16):
                av = _vgather(al, sidx[k])
                r = 16 * j + k
                for t in range(3):
                    hrs[t][r] = hrs[t][r] * av
            return 0
        lax.fori_loop(0, CHUNK // 16, inner, 0)

        for k in range(NSUB):
            for t in range(3):
                pltpu.sync_copy(hrs[t].at[pl.ds(SUB * k, SUB)],
                                oshs[t].at[dstv.at[k]], add=True)
        return 0
    lax.fori_loop(0, n_i, chunk_body, 0)

    plsc.subcore_barrier()
    for t in range(3):
        pltpu.sync_copy(oshs[t].at[pl.ds(sid * ROWS_PT, ROWS_PT)],
                        parts[t].at[cid, pl.ds(sid * ROWS_PT, ROWS_PT)])


def kernel(x, edge_index, W1, a1_src, a1_dst, W2, a2_src, a2_dst):
    f32 = jnp.float32
    i32 = jnp.int32
    src = edge_index[0].astype(i32)
    dst = edge_index[1].astype(i32)
    src2 = src.reshape(NCHUNKS, NSUB, SUB)
    dst2 = dst.reshape(NCHUNKS, NSUB, SUB)

    eye = jnp.eye(NH1, dtype=f32)
    ams = jnp.pad((a1_src[:, :, None] * eye[:, None, :]).reshape(D1, NH1),
                  ((0, 0), (0, 8)))
    amd = jnp.pad((a1_dst[:, :, None] * eye[:, None, :]).reshape(D1, NH1),
                  ((0, 0), (0, 8)))

    h1c0, h1c1, h1c2, h1c3, as16, ad16, g8 = pl.pallas_call(
        _tc1_body,
        out_shape=(
            jax.ShapeDtypeStruct((NB, 16), f32),
            jax.ShapeDtypeStruct((NB, 16), f32),
            jax.ShapeDtypeStruct((NB, 16), f32),
            jax.ShapeDtypeStruct((NB, 16), f32),
            jax.ShapeDtypeStruct((NB, 16), f32),
            jax.ShapeDtypeStruct((NB, 16), f32),
            jax.ShapeDtypeStruct((1, 16), f32),
        ),
    )(x, W1, ams, amd)
    g16 = jnp.where(jnp.arange(16) < NH1, g8[0], 40.0)

    mesh = plsc.VectorSubcoreMesh(core_axis_name="c", subcore_axis_name="s")
    idx_t = pltpu.VMEM((NSUB, SUB), i32)
    sc_params = pltpu.CompilerParams(use_tc_tiling_on_sc=False, needs_layout_passes=False)

    eexp1, dpart = pl.kernel(
        _sc_a1_body,
        out_type=(
            jax.ShapeDtypeStruct((EB, 16), f32),
            jax.ShapeDtypeStruct((2, NBP, 16), f32),
        ),
        mesh=mesh,
        compiler_params=sc_params,
        scratch_types=[
            idx_t, idx_t,
            pltpu.VMEM((CHUNK, 16), f32),
            pltpu.VMEM((CHUNK, 16), f32),
            pltpu.VMEM((CHUNK, 16), f32),
            pltpu.VMEM((16,), f32),
            pltpu.VMEM_SHARED((NBP, 16), f32),
            pltpu.SemaphoreType.DMA,
        ],
    )(src2, dst2, as16, ad16, g16)

    dinv1 = pl.pallas_call(
        _tc2_body, out_shape=jax.ShapeDtypeStruct((NBP, 16), f32),
    )(dpart)

    op0, op1, op2, op3 = pl.kernel(
        _sc_b1_body,
        out_type=tuple(jax.ShapeDtypeStruct((2, NBP, 16), f32) for _ in range(4)),
        mesh=mesh,
        compiler_params=sc_params,
        scratch_types=[
            idx_t, idx_t,
            pltpu.VMEM((CHUNK, 16), f32),
            pltpu.VMEM((CHUNK, 16), f32),
            pltpu.VMEM((CHUNK, 16), f32),
            pltpu.VMEM((CHUNK, 16), f32),
            pltpu.VMEM((CHUNK, 16), f32),
            pltpu.VMEM((CHUNK, 16), f32),
            pltpu.VMEM_SHARED((NBP, 16), f32),
            pltpu.VMEM_SHARED((NBP, 16), f32),
            pltpu.VMEM_SHARED((NBP, 16), f32),
            pltpu.VMEM_SHARED((NBP, 16), f32),
            pltpu.SemaphoreType.DMA,
        ],
    )(src2, dst2, h1c0, h1c1, h1c2, h1c3, eexp1, dinv1)

    w2p = jnp.pad(W2, ((0, 0), (0, D2 - NC2)))
    as2v = jnp.pad(a2_src[0], (0, D2 - NC2)).reshape(D2, 1)
    ad2v = jnp.pad(a2_dst[0], (0, D2 - NC2)).reshape(D2, 1)

    b3 = 1024
    pspec = pl.BlockSpec((2, b3, 16), lambda i: (0, i, 0))
    h2c0, h2c1, h2c2, s2, d2, g2 = pl.pallas_call(
        _tc3_body,
        grid=(NBP // b3,),
        in_specs=[pspec, pspec, pspec, pspec,
                  pl.BlockSpec((D1, D2), lambda i: (0, 0)),
                  pl.BlockSpec((D2, 1), lambda i: (0, 0)),
                  pl.BlockSpec((D2, 1), lambda i: (0, 0))],
        out_specs=(
            pl.BlockSpec((b3, 16), lambda i: (i, 0)),
            pl.BlockSpec((b3, 16), lambda i: (i, 0)),
            pl.BlockSpec((b3, 16), lambda i: (i, 0)),
            pl.BlockSpec((b3, 1), lambda i: (i, 0)),
            pl.BlockSpec((b3, 1), lambda i: (i, 0)),
            pl.BlockSpec((1, 1), lambda i: (0, 0)),
        ),
        out_shape=(
            jax.ShapeDtypeStruct((NBP, 16), f32),
            jax.ShapeDtypeStruct((NBP, 16), f32),
            jax.ShapeDtypeStruct((NBP, 16), f32),
            jax.ShapeDtypeStruct((NBP, 1), f32),
            jax.ShapeDtypeStruct((NBP, 1), f32),
            jax.ShapeDtypeStruct((1, 1), f32),
        ),
    )(op0, op1, op2, op3, w2p, as2v, ad2v)
    g2_16 = jnp.broadcast_to(g2[0, 0], (16,))

    eexp2, d2part = pl.kernel(
        _sc_a2_body,
        out_type=(
            jax.ShapeDtypeStruct((EB // 16, 16), f32),
            jax.ShapeDtypeStruct((2, NBP, 16), f32),
        ),
        mesh=mesh,
        compiler_params=sc_params,
        scratch_types=[
            idx_t, idx_t,
            pltpu.VMEM((NBP,), f32),
            pltpu.VMEM((NBP,), f32),
            pltpu.VMEM((CHUNK // 16, 16), f32),
            pltpu.VMEM((CHUNK, 16), f32),
            pltpu.VMEM((16,), f32),
            pltpu.VMEM_SHARED((NBP, 16), f32),
            pltpu.SemaphoreType.DMA,
        ],
    )(src2, dst2, s2.reshape(NBP), d2.reshape(NBP), g2_16)

    dinv2 = pl.pallas_call(
        _tc2_body, out_shape=jax.ShapeDtypeStruct((NBP, 16), f32),
    )(d2part)

    o2p0, o2p1, o2p2 = pl.kernel(
        _sc_b2_body,
        out_type=tuple(jax.ShapeDtypeStruct((2, NBP, 16), f32) for _ in range(3)),
        mesh=mesh,
        compiler_params=sc_params,
        scratch_types=[
            idx_t, idx_t,
            pltpu.VMEM((CHUNK, 16), f32),
            pltpu.VMEM((CHUNK, 16), f32),
            pltpu.VMEM((CHUNK, 16), f32),
            pltpu.VMEM((CHUNK // 16, 16), f32),
            pltpu.VMEM((NBP,), f32),
            pltpu.VMEM_SHARED((NBP, 16), f32),
            pltpu.VMEM_SHARED((NBP, 16), f32),
            pltpu.VMEM_SHARED((NBP, 16), f32),
            pltpu.SemaphoreType.DMA,
        ],
    )(src2, dst2, h2c0, h2c1, h2c2, eexp2, dinv2[:, 0])

    b5 = 1000
    pspec5 = pl.BlockSpec((2, b5, 16), lambda i: (0, i, 0))
    out = pl.pallas_call(
        _tc5_body,
        grid=(NB // b5,),
        in_specs=[pspec5, pspec5, pspec5],
        out_specs=pl.BlockSpec((b5, NC2), lambda i: (i, 0)),
        out_shape=jax.ShapeDtypeStruct((NB, NC2), f32),
    )(o2p0, o2p1, o2p2)
    return out


# speedup vs baseline: 83.1578x; 1.2917x over previous
import jax
import jax.numpy as jnp
from jax import lax
from jax.experimental import pallas as pl
from jax.experimental.pallas import tpu as pltpu
from jax.experimental.pallas import tpu_sc as plsc

NB = 10000
EB = 320000
NH1 = 8
D1 = 64
NC2 = 40
D2 = 48

CHUNK = 512
SUB = 128
NSUB = CHUNK // SUB
NCHUNKS = EB // CHUNK
NWORK = 32
NBP = 10240
ROWS_PT = NBP // 16


def _leaky(x):
    return jnp.maximum(x, 0.2 * x)


_GDN = lax.GatherDimensionNumbers(
    offset_dims=(), collapsed_slice_dims=(0,), start_index_map=(0,))


def _vgather(v, idx16):
    return lax.gather(v, idx16[:, None], _GDN, (1,),
                      mode=lax.GatherScatterMode.PROMISE_IN_BOUNDS)


def _tc1_body(x_ref, w1_ref, ams_ref, amd_ref, h_ref, as_ref, ad_ref, g_ref):
    h = jnp.dot(x_ref[...], w1_ref[...], preferred_element_type=jnp.float32)
    h_ref[...] = h
    as_ref[...] = jnp.dot(h, ams_ref[...], preferred_element_type=jnp.float32)
    ad_ref[...] = jnp.dot(h, amd_ref[...], preferred_element_type=jnp.float32)
    g_ref[...] = jnp.max(as_ref[...], axis=0, keepdims=True)


def _tc2_body(dp_ref, dinv_ref):
    dinv_ref[...] = 1.0 / (dp_ref[0] + dp_ref[1] + 1e-16)


def _tc3_body(p_ref, w2_ref, as2_ref, ad2_ref,
              h2_ref, s2_ref, d2_ref, g2_ref):
    i = pl.program_id(0)
    o = p_ref[0] + p_ref[1]
    hact = jnp.where(o > 0, o, jnp.exp(o) - 1.0)
    h2 = jnp.dot(hact, w2_ref[...], preferred_element_type=jnp.float32)
    h2_ref[...] = h2
    s2 = jnp.dot(h2, as2_ref[...], preferred_element_type=jnp.float32)
    d2 = jnp.dot(h2, ad2_ref[...], preferred_element_type=jnp.float32)
    s2_ref[...] = s2
    d2_ref[...] = d2
    m = jnp.max(s2, axis=0, keepdims=True)

    @pl.when(i == 0)
    def _():
        g2_ref[...] = m

    @pl.when(i > 0)
    def _():
        g2_ref[...] = jnp.maximum(g2_ref[...], m)


def _tc5_body(p_ref, out_ref):
    o = p_ref[0] + p_ref[1]
    t = o[:, :NC2]
    m = jnp.max(t, axis=1, keepdims=True)
    lse = jnp.log(jnp.sum(jnp.exp(t - m), axis=1, keepdims=True))
    out_ref[...] = t - m - lse


def _wid_and_niter():
    cid = lax.axis_index("c")
    sid = lax.axis_index("s")
    wid = sid * 2 + cid
    n_i = (NCHUNKS - wid + NWORK - 1) // NWORK
    return cid, sid, wid, n_i


def _zero_shared_slice(zbuf, shared, sid):
    pltpu.sync_copy(zbuf.at[pl.ds(0, CHUNK)],
                    shared.at[pl.ds(sid * ROWS_PT, CHUNK)])
    pltpu.sync_copy(zbuf.at[pl.ds(0, ROWS_PT - CHUNK)],
                    shared.at[pl.ds(sid * ROWS_PT + CHUNK, ROWS_PT - CHUNK)])


def _idx16(ref, j):
    return ref[j // (SUB // 16), pl.ds(16 * (j % (SUB // 16)), 16)]


def _sc_a1_body(src_hbm, dst_hbm, as_hbm, ad_hbm, g_hbm,
                eexp_hbm, dpart_hbm,
                srcv, dstv, asv, adv, eev, gv, dsh, sem):
    cid, sid, wid, n_i = _wid_and_niter()
    zero = jnp.zeros((16,), jnp.float32)

    def zb(j, _):
        eev[j] = zero
        return 0
    lax.fori_loop(0, CHUNK, zb, 0)
    _zero_shared_slice(eev, dsh, sid)
    pltpu.sync_copy(g_hbm, gv)
    plsc.subcore_barrier()

    g = gv[...]

    def chunk_body(i, _):
        c = wid + NWORK * i
        d1 = pltpu.async_copy(src_hbm.at[c], srcv, sem)
        d2 = pltpu.async_copy(dst_hbm.at[c], dstv, sem)
        d1.wait()
        d2.wait()
        ds_ = []
        for k in range(NSUB):
            ds_.append(pltpu.async_copy(
                as_hbm.at[srcv.at[k]], asv.at[pl.ds(SUB * k, SUB)], sem))
            ds_.append(pltpu.async_copy(
                ad_hbm.at[dstv.at[k]], adv.at[pl.ds(SUB * k, SUB)], sem))
        for d in ds_:
            d.wait()

        def inner(r, _):
            s = asv[r]
            a = adv[r]
            e = _leaky(s + a)
            cb = _leaky(g + a)
            eev[r] = jnp.exp(e - cb)
            return 0
        lax.fori_loop(0, CHUNK, inner, 0)

        for k in range(NSUB):
            pltpu.sync_copy(eev.at[pl.ds(SUB * k, SUB)], dsh.at[dstv.at[k]],
                            add=True)
        pltpu.sync_copy(eev, eexp_hbm.at[pl.ds(CHUNK * c, CHUNK)])
        return 0
    lax.fori_loop(0, n_i, chunk_body, 0)

    plsc.subcore_barrier()
    pltpu.sync_copy(dsh.at[pl.ds(sid * ROWS_PT, ROWS_PT)],
                    dpart_hbm.at[cid, pl.ds(sid * ROWS_PT, ROWS_PT)])


def _sc_b1_body(src_hbm, dst_hbm, h_hbm, eexp_hbm, dinv_hbm,
                opart_hbm,
                srcv, dstv, hrows, eev, dvv, osh, sem):
    cid, sid, wid, n_i = _wid_and_niter()
    zero = jnp.zeros((16,), jnp.float32)

    def zb(j, _):
        for t in range(4):
            hrows[j, pl.ds(16 * t, 16)] = zero
        return 0
    lax.fori_loop(0, CHUNK, zb, 0)
    _zero_shared_slice(hrows, osh, sid)
    plsc.subcore_barrier()

    q = lax.iota(jnp.int32, 16)
    bidx = [2 * t + lax.shift_right_logical(q, 3) for t in range(4)]

    def chunk_body(i, _):
        c = wid + NWORK * i
        d1 = pltpu.async_copy(src_hbm.at[c], srcv, sem)
        d2 = pltpu.async_copy(dst_hbm.at[c], dstv, sem)
        d1.wait()
        d2.wait()
        ds_ = [pltpu.async_copy(eexp_hbm.at[pl.ds(CHUNK * c, CHUNK)], eev, sem)]
        for k in range(NSUB):
            ds_.append(pltpu.async_copy(
                h_hbm.at[srcv.at[k]], hrows.at[pl.ds(SUB * k, SUB)], sem))
            ds_.append(pltpu.async_copy(
                dinv_hbm.at[dstv.at[k]], dvv.at[pl.ds(SUB * k, SUB)], sem))
        for d in ds_:
            d.wait()

        def inner(r, _):
            al = eev[r] * dvv[r]
            for t in range(4):
                av = _vgather(al, bidx[t])
                hrows[r, pl.ds(16 * t, 16)] = hrows[r, pl.ds(16 * t, 16)] * av
            return 0
        lax.fori_loop(0, CHUNK, inner, 0)

        for k in range(NSUB):
            pltpu.sync_copy(hrows.at[pl.ds(SUB * k, SUB)], osh.at[dstv.at[k]],
                            add=True)
        return 0
    lax.fori_loop(0, n_i, chunk_body, 0)

    plsc.subcore_barrier()
    pltpu.sync_copy(osh.at[pl.ds(sid * ROWS_PT, ROWS_PT)],
                    opart_hbm.at[cid, pl.ds(sid * ROWS_PT, ROWS_PT)])


def _sc_a2_body(src_hbm, dst_hbm, as2_hbm, ad2_hbm, g2_hbm,
                eexp2_hbm, d2part_hbm,
                srcv, dstv, as2t, ad2t, eevc, eevw, gv, dsh2, sem):
    cid, sid, wid, n_i = _wid_and_niter()
    zero = jnp.zeros((16,), jnp.float32)

    def zb(j, _):
        eevw[j] = zero
        return 0
    lax.fori_loop(0, CHUNK, zb, 0)
    _zero_shared_slice(eevw, dsh2, sid)
    pltpu.sync_copy(as2_hbm, as2t)
    pltpu.sync_copy(ad2_hbm, ad2t)
    pltpu.sync_copy(g2_hbm, gv)
    plsc.subcore_barrier()

    g = gv[...]
    q = lax.iota(jnp.int32, 16)
    zcol = q * 0

    def chunk_body(i, _):
        c = wid + NWORK * i
        d1 = pltpu.async_copy(src_hbm.at[c], srcv, sem)
        d2 = pltpu.async_copy(dst_hbm.at[c], dstv, sem)
        d1.wait()
        d2.wait()

        def inner(j, _):
            s16 = _idx16(srcv, j)
            d16 = _idx16(dstv, j)
            a_s = plsc.load_gather(as2t, [s16])
            a_d = plsc.load_gather(ad2t, [d16])
            e = _leaky(a_s + a_d)
            cb = _leaky(g + a_d)
            ee = jnp.exp(e - cb)
            eevc[j] = ee
            plsc.store_scatter(eevw, [16 * j + q, zcol], ee)
            return 0
        lax.fori_loop(0, CHUNK // 16, inner, 0)

        for k in range(NSUB):
            pltpu.sync_copy(eevw.at[pl.ds(SUB * k, SUB)], dsh2.at[dstv.at[k]],
                            add=True)
        pltpu.sync_copy(eevc, eexp2_hbm.at[pl.ds(CHUNK // 16 * c, CHUNK // 16)])
        return 0
    lax.fori_loop(0, n_i, chunk_body, 0)

    plsc.subcore_barrier()
    pltpu.sync_copy(dsh2.at[pl.ds(sid * ROWS_PT, ROWS_PT)],
                    d2part_hbm.at[cid, pl.ds(sid * ROWS_PT, ROWS_PT)])


def _sc_b2_body(src_hbm, dst_hbm, h2_hbm, eexp2_hbm, dinv2_hbm,
                o2part_hbm,
                srcv, dstv, hrows, eevc, dv2t, osh2, sem):
    cid, sid, wid, n_i = _wid_and_niter()
    zero = jnp.zeros((16,), jnp.float32)

    def zb(j, _):
        for t in range(3):
            hrows[j, pl.ds(16 * t, 16)] = zero
        return 0
    lax.fori_loop(0, CHUNK, zb, 0)
    _zero_shared_slice(hrows, osh2, sid)
    pltpu.sync_copy(dinv2_hbm, dv2t)
    plsc.subcore_barrier()

    q = lax.iota(jnp.int32, 16)
    sidx = [q * 0 + k for k in range(16)]

    def chunk_body(i, _):
        c = wid + NWORK * i
        d1 = pltpu.async_copy(src_hbm.at[c], srcv, sem)
        d2 = pltpu.async_copy(dst_hbm.at[c], dstv, sem)
        d1.wait()
        d2.wait()
        ds_ = [pltpu.async_copy(
            eexp2_hbm.at[pl.ds(CHUNK // 16 * c, CHUNK // 16)], eevc, sem)]
        for k in range(NSUB):
            ds_.append(pltpu.async_copy(
                h2_hbm.at[srcv.at[k]], hrows.at[pl.ds(SUB * k, SUB)], sem))
        for d in ds_:
            d.wait()

        def inner(j, _):
            d16 = _idx16(dstv, j)
            al = eevc[j] * plsc.load_gather(dv2t, [d16])
            for k in range(16):
                av = _vgather(al, sidx[k])
                r = 16 * j + k
                for t in range(3):
                    hrows[r, pl.ds(16 * t, 16)] = (
                        hrows[r, pl.ds(16 * t, 16)] * av)
            return 0
        lax.fori_loop(0, CHUNK // 16, inner, 0)

        for k in range(NSUB):
            pltpu.sync_copy(hrows.at[pl.ds(SUB * k, SUB)], osh2.at[dstv.at[k]],
                            add=True)
        return 0
    lax.fori_loop(0, n_i, chunk_body, 0)

    plsc.subcore_barrier()
    pltpu.sync_copy(osh2.at[pl.ds(sid * ROWS_PT, ROWS_PT)],
                    o2part_hbm.at[cid, pl.ds(sid * ROWS_PT, ROWS_PT)])


def kernel(x, edge_index, W1, a1_src, a1_dst, W2, a2_src, a2_dst):
    f32 = jnp.float32
    i32 = jnp.int32
    src = edge_index[0].astype(i32)
    dst = edge_index[1].astype(i32)
    src2 = src.reshape(NCHUNKS, NSUB, SUB)
    dst2 = dst.reshape(NCHUNKS, NSUB, SUB)

    eye = jnp.eye(NH1, dtype=f32)
    ams = jnp.pad((a1_src[:, :, None] * eye[:, None, :]).reshape(D1, NH1),
                  ((0, 0), (0, 8)))
    amd = jnp.pad((a1_dst[:, :, None] * eye[:, None, :]).reshape(D1, NH1),
                  ((0, 0), (0, 8)))

    h1, as16, ad16, g8 = pl.pallas_call(
        _tc1_body,
        out_shape=(
            jax.ShapeDtypeStruct((NB, D1), f32),
            jax.ShapeDtypeStruct((NB, 16), f32),
            jax.ShapeDtypeStruct((NB, 16), f32),
            jax.ShapeDtypeStruct((1, 16), f32),
        ),
    )(x, W1, ams, amd)
    g16 = jnp.where(jnp.arange(16) < NH1, g8[0], 40.0)

    mesh = plsc.VectorSubcoreMesh(core_axis_name="c", subcore_axis_name="s")
    idx_t = pltpu.VMEM((NSUB, SUB), i32)
    sc_params = pltpu.CompilerParams(use_tc_tiling_on_sc=False,
                                     needs_layout_passes=False)

    eexp1, dpart = pl.kernel(
        _sc_a1_body,
        out_type=(
            jax.ShapeDtypeStruct((EB, 16), f32),
            jax.ShapeDtypeStruct((2, NBP, 16), f32),
        ),
        mesh=mesh,
        compiler_params=sc_params,
        scratch_types=[
            idx_t, idx_t,
            pltpu.VMEM((CHUNK, 16), f32),
            pltpu.VMEM((CHUNK, 16), f32),
            pltpu.VMEM((CHUNK, 16), f32),
            pltpu.VMEM((16,), f32),
            pltpu.VMEM_SHARED((NBP, 16), f32),
            pltpu.SemaphoreType.DMA,
        ],
    )(src2, dst2, as16, ad16, g16)

    dinv1 = pl.pallas_call(
        _tc2_body, out_shape=jax.ShapeDtypeStruct((NBP, 16), f32),
    )(dpart)

    opart = pl.kernel(
        _sc_b1_body,
        out_type=jax.ShapeDtypeStruct((2, NBP, D1), f32),
        mesh=mesh,
        compiler_params=sc_params,
        scratch_types=[
            idx_t, idx_t,
            pltpu.VMEM((CHUNK, D1), f32),
            pltpu.VMEM((CHUNK, 16), f32),
            pltpu.VMEM((CHUNK, 16), f32),
            pltpu.VMEM_SHARED((NBP, D1), f32),
            pltpu.SemaphoreType.DMA,
        ],
    )(src2, dst2, h1, eexp1, dinv1)

    w2p = jnp.pad(W2, ((0, 0), (0, D2 - NC2)))
    as2v = jnp.pad(a2_src[0], (0, D2 - NC2)).reshape(D2, 1)
    ad2v = jnp.pad(a2_dst[0], (0, D2 - NC2)).reshape(D2, 1)

    b3 = 1024
    h2, s2, d2, g2 = pl.pallas_call(
        _tc3_body,
        grid=(NBP // b3,),
        in_specs=[pl.BlockSpec((2, b3, D1), lambda i: (0, i, 0)),
                  pl.BlockSpec((D1, D2), lambda i: (0, 0)),
                  pl.BlockSpec((D2, 1), lambda i: (0, 0)),
                  pl.BlockSpec((D2, 1), lambda i: (0, 0))],
        out_specs=(
            pl.BlockSpec((b3, D2), lambda i: (i, 0)),
            pl.BlockSpec((b3, 1), lambda i: (i, 0)),
            pl.BlockSpec((b3, 1), lambda i: (i, 0)),
            pl.BlockSpec((1, 1), lambda i: (0, 0)),
        ),
        out_shape=(
            jax.ShapeDtypeStruct((NBP, D2), f32),
            jax.ShapeDtypeStruct((NBP, 1), f32),
            jax.ShapeDtypeStruct((NBP, 1), f32),
            jax.ShapeDtypeStruct((1, 1), f32),
        ),
    )(opart, w2p, as2v, ad2v)
    g2_16 = jnp.broadcast_to(g2[0, 0], (16,))

    eexp2, d2part = pl.kernel(
        _sc_a2_body,
        out_type=(
            jax.ShapeDtypeStruct((EB // 16, 16), f32),
            jax.ShapeDtypeStruct((2, NBP, 16), f32),
        ),
        mesh=mesh,
        compiler_params=sc_params,
        scratch_types=[
            idx_t, idx_t,
            pltpu.VMEM((NBP,), f32),
            pltpu.VMEM((NBP,), f32),
            pltpu.VMEM((CHUNK // 16, 16), f32),
            pltpu.VMEM((CHUNK, 16), f32),
            pltpu.VMEM((16,), f32),
            pltpu.VMEM_SHARED((NBP, 16), f32),
            pltpu.SemaphoreType.DMA,
        ],
    )(src2, dst2, s2.reshape(NBP), d2.reshape(NBP), g2_16)

    dinv2 = pl.pallas_call(
        _tc2_body, out_shape=jax.ShapeDtypeStruct((NBP, 16), f32),
    )(d2part)

    o2part = pl.kernel(
        _sc_b2_body,
        out_type=jax.ShapeDtypeStruct((2, NBP, D2), f32),
        mesh=mesh,
        compiler_params=sc_params,
        scratch_types=[
            idx_t, idx_t,
            pltpu.VMEM((CHUNK, D2), f32),
            pltpu.VMEM((CHUNK // 16, 16), f32),
            pltpu.VMEM((NBP,), f32),
            pltpu.VMEM_SHARED((NBP, D2), f32),
            pltpu.SemaphoreType.DMA,
        ],
    )(src2, dst2, h2, eexp2, dinv2[:, 0])

    b5 = 1000
    out = pl.pallas_call(
        _tc5_body,
        grid=(NB // b5,),
        in_specs=[pl.BlockSpec((2, b5, D2), lambda i: (0, i, 0))],
        out_specs=pl.BlockSpec((b5, NC2), lambda i: (i, 0)),
        out_shape=jax.ShapeDtypeStruct((NB, NC2), f32),
    )(o2part)
    return out
